# baseline (device time: 34126 ns/iter reference)
import jax
import jax.numpy as jnp
from jax import lax
from jax.experimental import pallas as pl
from jax.experimental.pallas import tpu as pltpu

N_DEV = 32
E_LOC = 4
N_TOK = 1024
D_IN = 512
D_OUT = 1024
N_EXP = 128
ROWS_PER = N_TOK // N_DEV
CAP = 16
N_GRP = 4
GRP_DSTS = N_DEV // N_GRP
Q_TOT = N_DEV * CAP
Q_GRP = GRP_DSTS * CAP


def kernel(x, router_W, route_idx, expert_W):
    def body(x_ref, rw_ref, idx_ref, ew_ref, out_ref,
             send_buf, recv_buf, send_sems, recv_sems):
        my = lax.axis_index("i")

        barrier = pltpu.get_barrier_semaphore()
        for t in range(1, N_DEV):
            pl.semaphore_signal(
                barrier, inc=1,
                device_id=((my + t) % N_DEV,),
                device_id_type=pl.DeviceIdType.MESH)
        pl.semaphore_wait(barrier, N_DEV - 1)

        scores = jnp.dot(x_ref[:, :], rw_ref[:, :],
                         preferred_element_type=jnp.float32)
        smax = jnp.max(scores, axis=1, keepdims=True)
        p = jnp.exp(scores - smax)
        probs = p / jnp.sum(p, axis=1, keepdims=True)
        e0 = idx_ref[:, 0:1]
        e1 = idx_ref[:, 1:2]
        cols = lax.broadcasted_iota(jnp.int32, (N_TOK, N_EXP), 1)
        g0 = jnp.sum(jnp.where(cols == e0, probs, 0.0), axis=1, keepdims=True)
        g1 = jnp.sum(jnp.where(cols == e1, probs, 0.0), axis=1, keepdims=True)
        gs = g0 + g1
        g0n = g0 / gs
        g1n = g1 / gs

        dev0 = e0 // E_LOC
        dev1 = e1 // E_LOC
        hit = (dev0 == my) | (dev1 == my)
        hitb = jnp.where(hit, 1.0, 0.0).astype(jnp.bfloat16)
        ri = lax.broadcasted_iota(jnp.int32, (N_TOK, N_TOK), 0)
        ci = lax.broadcasted_iota(jnp.int32, (N_TOK, N_TOK), 1)
        m1 = jnp.where((ri // ROWS_PER == ci // ROWS_PER) & (ci < ri),
                       1.0, 0.0).astype(jnp.bfloat16)
        rank = jnp.dot(m1, hitb, preferred_element_type=jnp.float32)

        tq = lax.broadcasted_iota(jnp.int32, (N_TOK, Q_TOT), 1)
        tt = lax.broadcasted_iota(jnp.int32, (N_TOK, Q_TOT), 0)
        jq = (tq // CAP + GRP_DSTS * (my // GRP_DSTS)) % N_DEV
        cq = (tq % CAP).astype(jnp.float32)
        gt = jnp.where(hit & (tt // ROWS_PER == jq) & (rank == cq),
                       1.0, 0.0).astype(jnp.bfloat16)

        ki = lax.broadcasted_iota(jnp.int32, (N_TOK, E_LOC), 1)
        eg = my * E_LOC + ki
        w4 = (jnp.where(e0 == eg, g0n, 0.0)
              + jnp.where(e1 == eg, g1n, 0.0)).astype(jnp.bfloat16)

        xb = x_ref[:, :].astype(jnp.bfloat16)
        weights = [ew_ref[k].astype(jnp.bfloat16) for k in range(E_LOC)]
        cdims = (((0,), (0,)), ((), ()))

        for gg in range(N_GRP):
            gt_g = gt[:, gg * Q_GRP:(gg + 1) * Q_GRP]
            xg = lax.dot_general(gt_g, xb, cdims,
                                 preferred_element_type=jnp.float32)
            gw = lax.dot_general(gt_g, w4, cdims,
                                 preferred_element_type=jnp.float32)
            acc = jnp.zeros((Q_GRP, D_OUT), jnp.float32)
            for k in range(E_LOC):
                xwk = (xg * gw[:, k:k + 1]).astype(jnp.bfloat16)
                acc = acc + jnp.dot(xwk, weights[k],
                                    preferred_element_type=jnp.float32)
            send_buf[gg * Q_GRP:(gg + 1) * Q_GRP, :] = acc.astype(jnp.bfloat16)

            for u in range(GRP_DSTS):
                qd = gg * GRP_DSTS + u
                j = (qd + GRP_DSTS * (my // GRP_DSTS)) % N_DEV

                @pl.when(j != my)
                def _send(qd=qd, j=j):
                    pltpu.make_async_remote_copy(
                        src_ref=send_buf.at[pl.ds(qd * CAP, CAP), :],
                        dst_ref=recv_buf.at[pl.ds(my * CAP, CAP), :],
                        send_sem=send_sems.at[qd],
                        recv_sem=recv_sems.at[my],
                        device_id=(j,),
                        device_id_type=pl.DeviceIdType.MESH,
                    ).start()

                @pl.when(j == my)
                def _own(qd=qd):
                    recv_buf[pl.ds(my * CAP, CAP), :] = (
                        send_buf[pl.ds(qd * CAP, CAP), :])

        e0r = idx_ref[pl.ds(my * ROWS_PER, ROWS_PER), 0:1]
        e1r = idx_ref[pl.ds(my * ROWS_PER, ROWS_PER), 1:2]
        d0r = e0r // E_LOC
        d1r = e1r // E_LOC
        ql = lax.broadcasted_iota(jnp.int32, (ROWS_PER, Q_TOT), 1)
        s_l = ql // CAP
        c_l = (ql % CAP).astype(jnp.float32)
        hit_d = (d0r == s_l) | (d1r == s_l)
        hdb = jnp.where(hit_d, 1.0, 0.0).astype(jnp.bfloat16)
        ri3 = lax.broadcasted_iota(jnp.int32, (ROWS_PER, ROWS_PER), 0)
        ci3 = lax.broadcasted_iota(jnp.int32, (ROWS_PER, ROWS_PER), 1)
        lstrict = jnp.where(ci3 < ri3, 1.0, 0.0).astype(jnp.bfloat16)
        rank_d = jnp.dot(lstrict, hdb, preferred_element_type=jnp.float32)
        pd = jnp.where(hit_d & (rank_d == c_l),
                       1.0, 0.0).astype(jnp.bfloat16)

        for t in range(1, N_DEV):
            s = (my + t) % N_DEV
            pltpu.make_async_remote_copy(
                src_ref=send_buf.at[pl.ds(s * CAP, CAP), :],
                dst_ref=recv_buf.at[pl.ds(s * CAP, CAP), :],
                send_sem=send_sems.at[s],
                recv_sem=recv_sems.at[s],
                device_id=(s,),
                device_id_type=pl.DeviceIdType.MESH,
            ).wait_recv()

        out_ref[:, :] = jnp.dot(pd, recv_buf[:, :],
                                preferred_element_type=jnp.float32)

        for qd in range(N_DEV):
            j = (qd + GRP_DSTS * (my // GRP_DSTS)) % N_DEV

            @pl.when(j != my)
            def _drain(qd=qd, j=j):
                pltpu.make_async_remote_copy(
                    src_ref=send_buf.at[pl.ds(qd * CAP, CAP), :],
                    dst_ref=recv_buf.at[pl.ds(my * CAP, CAP), :],
                    send_sem=send_sems.at[qd],
                    recv_sem=recv_sems.at[my],
                    device_id=(j,),
                    device_id_type=pl.DeviceIdType.MESH,
                ).wait_send()

    return pl.pallas_call(
        body,
        out_shape=jax.ShapeDtypeStruct((ROWS_PER, D_OUT), jnp.float32),
        in_specs=[pl.BlockSpec(memory_space=pltpu.VMEM)] * 4,
        out_specs=pl.BlockSpec(memory_space=pltpu.VMEM),
        scratch_shapes=[
            pltpu.VMEM((Q_TOT, D_OUT), jnp.bfloat16),
            pltpu.VMEM((Q_TOT, D_OUT), jnp.bfloat16),
            pltpu.SemaphoreType.DMA((N_DEV,)),
            pltpu.SemaphoreType.DMA((N_DEV,)),
        ],
        compiler_params=pltpu.CompilerParams(collective_id=0),
    )(x, router_W, route_idx, expert_W)
